# Initial kernel scaffold; baseline (speedup 1.0000x reference)
import jax
import jax.numpy as jnp
from jax.experimental import pallas as pl


def kernel(labels, train, force_drop_mask, embedding_table):
    raise NotImplementedError("write your pallas kernel here")


# speedup vs baseline: 1.0437x; 1.0437x over previous
import functools

import jax
import jax.numpy as jnp
from jax import lax
from jax.experimental import pallas as pl
from jax.experimental.pallas import tpu as pltpu
from jax.experimental.pallas import tpu_sc as plsc

_NULL_ROW = 100000
_CHUNK = 128


@functools.lru_cache(maxsize=None)
def _make_kernel(B, D):
    info = plsc.get_sparse_core_info()
    nc, ns = info.num_cores, info.num_subcores
    nw = nc * ns
    b_per_w = B // nw
    n_chunks = b_per_w // _CHUNK
    mesh = plsc.VectorSubcoreMesh(core_axis_name="c", subcore_axis_name="s")

    @functools.partial(
        pl.kernel,
        mesh=mesh,
        out_type=jax.ShapeDtypeStruct((B, D), jnp.float32),
        scratch_types=[
            pltpu.VMEM((n_chunks, _CHUNK), jnp.int32),
            pltpu.VMEM((n_chunks, _CHUNK), jnp.int32),
            pltpu.VMEM((b_per_w, D), jnp.float32),
            pltpu.SemaphoreType.DMA,
        ],
    )
    def k(labels_hbm, mask_hbm, table_hbm, out_hbm, idx_v, msk_v, rows_v, sem):
        wid = lax.axis_index("s") * nc + lax.axis_index("c")
        row0 = wid * n_chunks
        pltpu.sync_copy(labels_hbm.at[pl.ds(row0, n_chunks)], idx_v)
        pltpu.sync_copy(mask_hbm.at[pl.ds(row0, n_chunks)], msk_v)
        null_v = jnp.full((16,), _NULL_ROW, jnp.int32)
        for j in range(n_chunks):
            for c in range(_CHUNK // 16):
                sl = pl.ds(c * 16, 16)
                idx_v[j, sl] = jnp.where(msk_v[j, sl] != 0, null_v, idx_v[j, sl])
        copies = [
            pltpu.async_copy(
                table_hbm.at[idx_v.at[j]],
                rows_v.at[pl.ds(j * _CHUNK, _CHUNK)],
                sem,
            )
            for j in range(n_chunks)
        ]
        for cp in copies:
            cp.wait()
        pltpu.sync_copy(rows_v, out_hbm.at[pl.ds(wid * b_per_w, b_per_w)])

    return k


def kernel(labels, train, force_drop_mask, embedding_table):
    del train
    (B,) = labels.shape
    _, D = embedding_table.shape
    labels2 = labels.astype(jnp.int32).reshape(B // _CHUNK, _CHUNK)
    mask2 = force_drop_mask.astype(jnp.int32).reshape(B // _CHUNK, _CHUNK)
    return _make_kernel(B, D)(labels2, mask2, embedding_table)
